# baseline (speedup 1.0000x reference)
import jax
import jax.numpy as jnp
from jax.experimental import pallas as pl
from jax.experimental.pallas import tpu as pltpu

_VMEM_LIMIT_BYTES = 48 * 1024 * 1024


def kernel(x, w1, w2):
    B, C, H, W = x.shape
    HW = H * W
    hidden = w1.shape[0]
    inv_hw = 1.0 / float(HW)

    x3 = x.reshape(B, C, HW)

    def body(x_ref, w1_ref, w2_ref, o_ref):
        xb = x_ref[0]
        pooled = jnp.sum(xb, axis=-1, keepdims=True,
                         dtype=jnp.float32) * inv_hw
        h = jnp.maximum(
            jnp.dot(w1_ref[...], pooled,
                    preferred_element_type=jnp.float32), 0.0)
        gate = jax.nn.sigmoid(
            jnp.dot(w2_ref[...], h,
                    preferred_element_type=jnp.float32))
        o_ref[0] = (xb * gate.astype(xb.dtype)).astype(o_ref.dtype)

    out = pl.pallas_call(
        body,
        out_shape=jax.ShapeDtypeStruct((B, C, HW), x.dtype),
        grid=(B,),
        in_specs=[
            pl.BlockSpec((1, C, HW), lambda b: (b, 0, 0)),
            pl.BlockSpec((hidden, C), lambda b: (0, 0)),
            pl.BlockSpec((C, hidden), lambda b: (0, 0)),
        ],
        out_specs=pl.BlockSpec((1, C, HW), lambda b: (b, 0, 0)),
        compiler_params=pltpu.CompilerParams(
            dimension_semantics=("parallel",),
            vmem_limit_bytes=_VMEM_LIMIT_BYTES),
    )(x3, w1, w2)
    return out.reshape(B, C, H, W)


# speedup vs baseline: 1.4272x; 1.4272x over previous
import jax
import jax.numpy as jnp
from jax.experimental import pallas as pl
from jax.experimental.pallas import tpu as pltpu

_VMEM_LIMIT_BYTES = 48 * 1024 * 1024


def kernel(x, w1, w2):
    B, C, H, W = x.shape
    HW = H * W
    hidden = w1.shape[0]
    inv_hw = 1.0 / float(HW)

    x3 = x.reshape(B, C, HW)

    NB = 8
    assert B % NB == 0

    def body(x_ref, w1_ref, w2_ref, o_ref):
        xb = x_ref[...].reshape(NB * C, HW)
        pooled = jnp.sum(xb, axis=-1, keepdims=True,
                         dtype=jnp.float32) * inv_hw
        gates = []
        for b in range(NB):
            pb = pooled[b * C:(b + 1) * C]
            hb = jnp.maximum(
                jnp.dot(w1_ref[...], pb,
                        preferred_element_type=jnp.float32), 0.0)
            gates.append(jnp.dot(w2_ref[...], hb,
                                 preferred_element_type=jnp.float32))
        gate = jax.nn.sigmoid(jnp.concatenate(gates, axis=0))
        o_ref[...] = (xb * gate.astype(xb.dtype)).reshape(
            NB, C, HW).astype(o_ref.dtype)

    out = pl.pallas_call(
        body,
        out_shape=jax.ShapeDtypeStruct((B, C, HW), x.dtype),
        grid=(B // NB,),
        in_specs=[
            pl.BlockSpec((NB, C, HW), lambda b: (b, 0, 0)),
            pl.BlockSpec((hidden, C), lambda b: (0, 0)),
            pl.BlockSpec((C, hidden), lambda b: (0, 0)),
        ],
        out_specs=pl.BlockSpec((NB, C, HW), lambda b: (b, 0, 0)),
        compiler_params=pltpu.CompilerParams(
            dimension_semantics=("parallel",),
            vmem_limit_bytes=_VMEM_LIMIT_BYTES),
    )(x3, w1, w2)
    return out.reshape(B, C, H, W)


# speedup vs baseline: 1.4396x; 1.0087x over previous
import jax
import jax.numpy as jnp
from jax.experimental import pallas as pl
from jax.experimental.pallas import tpu as pltpu

_VMEM_LIMIT_BYTES = 48 * 1024 * 1024


def kernel(x, w1, w2):
    B, C, H, W = x.shape
    HW = H * W
    hidden = w1.shape[0]
    inv_hw = 1.0 / float(HW)

    x3 = x.reshape(B, C, HW)

    NB = 16
    assert B % NB == 0

    def body(x_ref, w1_ref, w2_ref, o_ref):
        xb = x_ref[...].reshape(NB * C, HW)
        pooled = jnp.sum(xb, axis=-1, keepdims=True,
                         dtype=jnp.float32) * inv_hw
        gates = []
        for b in range(NB):
            pb = pooled[b * C:(b + 1) * C]
            hb = jnp.maximum(
                jnp.dot(w1_ref[...], pb,
                        preferred_element_type=jnp.float32), 0.0)
            gates.append(jnp.dot(w2_ref[...], hb,
                                 preferred_element_type=jnp.float32))
        gate = jax.nn.sigmoid(jnp.concatenate(gates, axis=0))
        o_ref[...] = (xb * gate.astype(xb.dtype)).reshape(
            NB, C, HW).astype(o_ref.dtype)

    out = pl.pallas_call(
        body,
        out_shape=jax.ShapeDtypeStruct((B, C, HW), x.dtype),
        grid=(B // NB,),
        in_specs=[
            pl.BlockSpec((NB, C, HW), lambda b: (b, 0, 0)),
            pl.BlockSpec((hidden, C), lambda b: (0, 0)),
            pl.BlockSpec((C, hidden), lambda b: (0, 0)),
        ],
        out_specs=pl.BlockSpec((NB, C, HW), lambda b: (b, 0, 0)),
        compiler_params=pltpu.CompilerParams(
            dimension_semantics=("parallel",),
            vmem_limit_bytes=_VMEM_LIMIT_BYTES),
    )(x3, w1, w2)
    return out.reshape(B, C, H, W)
